# baseline (speedup 1.0000x reference)
import functools

import jax
import jax.numpy as jnp
from jax import lax
from jax.experimental import pallas as pl
from jax.experimental.pallas import tpu as pltpu
from jax.experimental.pallas import tpu_sc as plsc

EMBED_DIM = 64
LANES = 16
NCOL = EMBED_DIM // LANES

NC, NS = 2, 16
NW = NC * NS
IBUF = 2


def _pooled_gather_body(idx_hbm, table_hbm, out_hbm, idx_v, rows_v, out_v,
                        *sems):
    bpw, n_docs, doc_len = idx_v.shape
    wid = lax.axis_index("s") * NC + lax.axis_index("c")
    base = wid * bpw
    nbuf = IBUF * n_docs

    pltpu.sync_copy(idx_hbm.at[pl.ds(base, bpw)], idx_v)

    for ii in range(IBUF):
        for j in range(n_docs):
            b = ii * n_docs + j
            pltpu.async_copy(
                table_hbm.at[idx_v.at[ii, j]], rows_v.at[b], sems[b])

    scale = jnp.float32(1.0 / doc_len)

    @pl.loop(0, bpw, step=IBUF)
    def _row(i0):
        for ii in range(IBUF):
            i = i0 + ii
            for j in range(n_docs):
                b = ii * n_docs + j
                pltpu.make_async_copy(
                    table_hbm.at[idx_v.at[i, j]], rows_v.at[b],
                    sems[b]).wait()

                def acc_body(l, accs, _b=b):
                    return tuple(
                        a + rows_v[_b, l, pl.ds(d * LANES, LANES)]
                        for d, a in enumerate(accs))

                accs = lax.fori_loop(
                    0, doc_len, acc_body,
                    tuple(jnp.zeros((LANES,), jnp.float32)
                          for _ in range(NCOL)),
                    unroll=10)
                for d in range(NCOL):
                    out_v[i, j, pl.ds(d * LANES, LANES)] = accs[d] * scale

                nxt = i + IBUF

                @pl.when(nxt < bpw)
                def _prefetch(_b=b, _j=j, _nxt=nxt):
                    pltpu.async_copy(
                        table_hbm.at[idx_v.at[_nxt, _j]], rows_v.at[_b],
                        sems[_b])

    pltpu.sync_copy(out_v, out_hbm.at[pl.ds(base, bpw)])


def kernel(numericalized_doc_toks, embedding):
    batch, n_docs, doc_len = numericalized_doc_toks.shape
    bpw = batch // NW
    nbuf = IBUF * n_docs

    mesh = plsc.VectorSubcoreMesh(core_axis_name="c", subcore_axis_name="s")
    run = functools.partial(
        pl.kernel,
        out_type=jax.ShapeDtypeStruct((batch, n_docs, EMBED_DIM),
                                      jnp.float32),
        mesh=mesh,
        scratch_types=[
            pltpu.VMEM((bpw, n_docs, doc_len), jnp.int32),
            pltpu.VMEM((nbuf, doc_len, EMBED_DIM), jnp.float32),
            pltpu.VMEM((bpw, n_docs, EMBED_DIM), jnp.float32),
        ] + [pltpu.SemaphoreType.DMA] * nbuf,
        compiler_params=pltpu.CompilerParams(use_tc_tiling_on_sc=False),
    )(_pooled_gather_body)
    vocab = embedding.shape[0]
    paired = lax.optimization_barrier(
        jnp.concatenate([embedding[0::2], embedding[1::2]], axis=1))
    table_rm = paired.reshape(vocab, EMBED_DIM)
    return run(numericalized_doc_toks, table_rm)


# speedup vs baseline: 11.8334x; 11.8334x over previous
import functools

import jax
import jax.numpy as jnp
from jax import lax
from jax.experimental import pallas as pl
from jax.experimental.pallas import tpu as pltpu
from jax.experimental.pallas import tpu_sc as plsc

EMBED_DIM = 64
LANES = 16
NCOL = EMBED_DIM // LANES

NC, NS = 2, 16
NW = NC * NS
IBUF = 2
CBLK = 1536


def _tc_transpose_body(et_ref, out_ref):
    y = et_ref[...].T
    z = y.reshape(y.shape[0] // 2, 2, y.shape[1])
    out_ref[...] = jnp.concatenate([z[:, 0, :], z[:, 1, :]], axis=1)


def _pooled_gather_body(idx_hbm, table_hbm, out_hbm, idx_v, rows_v, out_v,
                        *sems):
    bpw, n_docs, doc_len = idx_v.shape
    wid = lax.axis_index("s") * NC + lax.axis_index("c")
    base = wid * bpw

    pltpu.sync_copy(idx_hbm.at[pl.ds(base, bpw)], idx_v)

    for ii in range(IBUF):
        for j in range(n_docs):
            b = ii * n_docs + j
            pltpu.async_copy(
                table_hbm.at[idx_v.at[ii, j]], rows_v.at[b], sems[b])

    scale = jnp.float32(1.0 / doc_len)

    @pl.loop(0, bpw, step=IBUF)
    def _row(i0):
        for ii in range(IBUF):
            i = i0 + ii
            for j in range(n_docs):
                b = ii * n_docs + j
                pltpu.make_async_copy(
                    table_hbm.at[idx_v.at[i, j]], rows_v.at[b],
                    sems[b]).wait()

                def acc_body(l, accs, _b=b):
                    return tuple(
                        a + rows_v[_b, l, pl.ds(d * LANES, LANES)]
                        for d, a in enumerate(accs))

                accs = lax.fori_loop(
                    0, doc_len, acc_body,
                    tuple(jnp.zeros((LANES,), jnp.float32)
                          for _ in range(NCOL)),
                    unroll=10)
                for d in range(NCOL):
                    out_v[i, j, pl.ds(d * LANES, LANES)] = accs[d] * scale

                nxt = i + IBUF

                @pl.when(nxt < bpw)
                def _prefetch(_b=b, _j=j, _nxt=nxt):
                    pltpu.async_copy(
                        table_hbm.at[idx_v.at[_nxt, _j]], rows_v.at[_b],
                        sems[_b])

    pltpu.sync_copy(out_v, out_hbm.at[pl.ds(base, bpw)])


def kernel(numericalized_doc_toks, embedding):
    batch, n_docs, doc_len = numericalized_doc_toks.shape
    vocab = embedding.shape[0]
    bpw = batch // NW
    nbuf = IBUF * n_docs
    vpad = vocab

    mesh = plsc.VectorSubcoreMesh(core_axis_name="c", subcore_axis_name="s")

    cblk = CBLK
    n_cblk = (vpad + cblk - 1) // cblk
    relayout = pl.pallas_call(
        _tc_transpose_body,
        grid=(n_cblk,),
        in_specs=[pl.BlockSpec((EMBED_DIM, cblk), lambda k: (0, k))],
        out_specs=pl.BlockSpec((cblk // 2, 2 * EMBED_DIM), lambda k: (k, 0)),
        out_shape=jax.ShapeDtypeStruct((vpad // 2, 2 * EMBED_DIM),
                                       jnp.float32),
    )
    table_rm = relayout(embedding.T).reshape(vpad, EMBED_DIM)

    gather = functools.partial(
        pl.kernel,
        out_type=jax.ShapeDtypeStruct((batch, n_docs, EMBED_DIM),
                                      jnp.float32),
        mesh=mesh,
        scratch_types=[
            pltpu.VMEM((bpw, n_docs, doc_len), jnp.int32),
            pltpu.VMEM((nbuf, doc_len, EMBED_DIM), jnp.float32),
            pltpu.VMEM((bpw, n_docs, EMBED_DIM), jnp.float32),
        ] + [pltpu.SemaphoreType.DMA] * nbuf,
        compiler_params=pltpu.CompilerParams(use_tc_tiling_on_sc=False),
    )(_pooled_gather_body)
    return gather(numericalized_doc_toks, table_rm)


# speedup vs baseline: 16.5668x; 1.4000x over previous
import functools

import jax
import jax.numpy as jnp
from jax import lax
from jax.experimental import pallas as pl
from jax.experimental.pallas import tpu as pltpu
from jax.experimental.pallas import tpu_sc as plsc

EMBED_DIM = 64
LANES = 16
NCOL = EMBED_DIM // LANES

NC, NS = 2, 16
NW = NC * NS
IBUF = 2
CBLK = 6144


def _tc_transpose_body(et_ref, out_ref):
    y = et_ref[...].T
    z = y.reshape(y.shape[0] // 2, 2, y.shape[1])
    out_ref[...] = jnp.concatenate([z[:, 0, :], z[:, 1, :]], axis=1)


def _pooled_gather_body(idx_hbm, table_hbm, out_hbm, idx_v, rows_v, out_v,
                        *sems):
    bpw, n_docs, doc_len = idx_v.shape
    wid = lax.axis_index("s") * NC + lax.axis_index("c")
    base = wid * bpw

    pltpu.sync_copy(idx_hbm.at[pl.ds(base, bpw)], idx_v)

    for ii in range(IBUF):
        for j in range(n_docs):
            b = ii * n_docs + j
            pltpu.async_copy(
                table_hbm.at[idx_v.at[ii, j]], rows_v.at[b], sems[b])

    scale = jnp.float32(1.0 / doc_len)

    @pl.loop(0, bpw, step=IBUF)
    def _row(i0):
        for ii in range(IBUF):
            i = i0 + ii
            for j in range(n_docs):
                b = ii * n_docs + j
                pltpu.make_async_copy(
                    table_hbm.at[idx_v.at[i, j]], rows_v.at[b],
                    sems[b]).wait()

                def acc_body(l, accs, _b=b):
                    return tuple(
                        a + rows_v[_b, l, pl.ds(d * LANES, LANES)]
                        for d, a in enumerate(accs))

                accs = lax.fori_loop(
                    0, doc_len, acc_body,
                    tuple(jnp.zeros((LANES,), jnp.float32)
                          for _ in range(NCOL)),
                    unroll=10)
                for d in range(NCOL):
                    out_v[i, j, pl.ds(d * LANES, LANES)] = accs[d] * scale

                nxt = i + IBUF

                @pl.when(nxt < bpw)
                def _prefetch(_b=b, _j=j, _nxt=nxt):
                    pltpu.async_copy(
                        table_hbm.at[idx_v.at[_nxt, _j]], rows_v.at[_b],
                        sems[_b])

    pltpu.sync_copy(out_v, out_hbm.at[pl.ds(base, bpw)])


def kernel(numericalized_doc_toks, embedding):
    batch, n_docs, doc_len = numericalized_doc_toks.shape
    vocab = embedding.shape[0]
    bpw = batch // NW
    nbuf = IBUF * n_docs
    vpad = vocab

    mesh = plsc.VectorSubcoreMesh(core_axis_name="c", subcore_axis_name="s")

    cblk = CBLK
    n_cblk = (vpad + cblk - 1) // cblk
    relayout = pl.pallas_call(
        _tc_transpose_body,
        grid=(n_cblk,),
        in_specs=[pl.BlockSpec((EMBED_DIM, cblk), lambda k: (0, k))],
        out_specs=pl.BlockSpec((cblk // 2, 2 * EMBED_DIM), lambda k: (k, 0)),
        out_shape=jax.ShapeDtypeStruct((vpad // 2, 2 * EMBED_DIM),
                                       jnp.float32),
    )
    table_rm = relayout(embedding.T).reshape(vpad, EMBED_DIM)

    gather = functools.partial(
        pl.kernel,
        out_type=jax.ShapeDtypeStruct((batch, n_docs, EMBED_DIM),
                                      jnp.float32),
        mesh=mesh,
        scratch_types=[
            pltpu.VMEM((bpw, n_docs, doc_len), jnp.int32),
            pltpu.VMEM((nbuf, doc_len, EMBED_DIM), jnp.float32),
            pltpu.VMEM((bpw, n_docs, EMBED_DIM), jnp.float32),
        ] + [pltpu.SemaphoreType.DMA] * nbuf,
        compiler_params=pltpu.CompilerParams(use_tc_tiling_on_sc=False),
    )(_pooled_gather_body)
    return gather(numericalized_doc_toks, table_rm)


# speedup vs baseline: 16.7599x; 1.0117x over previous
import functools

import jax
import jax.numpy as jnp
from jax import lax
from jax.experimental import pallas as pl
from jax.experimental.pallas import tpu as pltpu
from jax.experimental.pallas import tpu_sc as plsc

EMBED_DIM = 64
LANES = 16
NCOL = EMBED_DIM // LANES

NC, NS = 2, 16
NW = NC * NS
IBUF = 2
CBLK = 12288


def _tc_transpose_body(et_ref, out_ref):
    y = et_ref[...].T
    z = y.reshape(y.shape[0] // 2, 2, y.shape[1])
    out_ref[...] = jnp.concatenate([z[:, 0, :], z[:, 1, :]], axis=1)


def _pooled_gather_body(idx_hbm, table_hbm, out_hbm, idx_v, rows_v, out_v,
                        *sems):
    bpw, n_docs, doc_len = idx_v.shape
    wid = lax.axis_index("s") * NC + lax.axis_index("c")
    base = wid * bpw

    pltpu.sync_copy(idx_hbm.at[pl.ds(base, bpw)], idx_v)

    for ii in range(IBUF):
        for j in range(n_docs):
            b = ii * n_docs + j
            pltpu.async_copy(
                table_hbm.at[idx_v.at[ii, j]], rows_v.at[b], sems[b])

    scale = jnp.float32(1.0 / doc_len)

    @pl.loop(0, bpw, step=IBUF)
    def _row(i0):
        for ii in range(IBUF):
            i = i0 + ii
            for j in range(n_docs):
                b = ii * n_docs + j
                pltpu.make_async_copy(
                    table_hbm.at[idx_v.at[i, j]], rows_v.at[b],
                    sems[b]).wait()

                def acc_body(l, accs, _b=b):
                    return tuple(
                        a + rows_v[_b, l, pl.ds(d * LANES, LANES)]
                        for d, a in enumerate(accs))

                accs = lax.fori_loop(
                    0, doc_len, acc_body,
                    tuple(jnp.zeros((LANES,), jnp.float32)
                          for _ in range(NCOL)),
                    unroll=10)
                for d in range(NCOL):
                    out_v[i, j, pl.ds(d * LANES, LANES)] = accs[d] * scale

                nxt = i + IBUF

                @pl.when(nxt < bpw)
                def _prefetch(_b=b, _j=j, _nxt=nxt):
                    pltpu.async_copy(
                        table_hbm.at[idx_v.at[_nxt, _j]], rows_v.at[_b],
                        sems[_b])

    pltpu.sync_copy(out_v, out_hbm.at[pl.ds(base, bpw)])


def kernel(numericalized_doc_toks, embedding):
    batch, n_docs, doc_len = numericalized_doc_toks.shape
    vocab = embedding.shape[0]
    bpw = batch // NW
    nbuf = IBUF * n_docs
    vpad = vocab

    mesh = plsc.VectorSubcoreMesh(core_axis_name="c", subcore_axis_name="s")

    cblk = CBLK
    n_cblk = (vpad + cblk - 1) // cblk
    relayout = pl.pallas_call(
        _tc_transpose_body,
        grid=(n_cblk,),
        in_specs=[pl.BlockSpec((EMBED_DIM, cblk), lambda k: (0, k))],
        out_specs=pl.BlockSpec((cblk // 2, 2 * EMBED_DIM), lambda k: (k, 0)),
        out_shape=jax.ShapeDtypeStruct((vpad // 2, 2 * EMBED_DIM),
                                       jnp.float32),
    )
    table_rm = relayout(embedding.T).reshape(vpad, EMBED_DIM)

    gather = functools.partial(
        pl.kernel,
        out_type=jax.ShapeDtypeStruct((batch, n_docs, EMBED_DIM),
                                      jnp.float32),
        mesh=mesh,
        scratch_types=[
            pltpu.VMEM((bpw, n_docs, doc_len), jnp.int32),
            pltpu.VMEM((nbuf, doc_len, EMBED_DIM), jnp.float32),
            pltpu.VMEM((bpw, n_docs, EMBED_DIM), jnp.float32),
        ] + [pltpu.SemaphoreType.DMA] * nbuf,
        compiler_params=pltpu.CompilerParams(use_tc_tiling_on_sc=False),
    )(_pooled_gather_body)
    return gather(numericalized_doc_toks, table_rm)


# speedup vs baseline: 16.8585x; 1.0059x over previous
import functools

import jax
import jax.numpy as jnp
from jax import lax
from jax.experimental import pallas as pl
from jax.experimental.pallas import tpu as pltpu
from jax.experimental.pallas import tpu_sc as plsc

EMBED_DIM = 64
LANES = 16
NCOL = EMBED_DIM // LANES

NC, NS = 2, 16
NW = NC * NS
IBUF = 2
CBLK = 12288
CSUB = 3072


def _tc_transpose_body(et_ref, out_ref):
    for c in range(CBLK // CSUB):
        y = et_ref[:, pl.ds(c * CSUB, CSUB)].T
        z = y.reshape(CSUB // 2, 2, EMBED_DIM)
        out_ref[pl.ds(c * CSUB // 2, CSUB // 2), :] = jnp.concatenate(
            [z[:, 0, :], z[:, 1, :]], axis=1)


def _pooled_gather_body(idx_hbm, table_hbm, out_hbm, idx_v, rows_v, out_v,
                        *sems):
    bpw, n_docs, doc_len = idx_v.shape
    wid = lax.axis_index("s") * NC + lax.axis_index("c")
    base = wid * bpw

    pltpu.sync_copy(idx_hbm.at[pl.ds(base, bpw)], idx_v)

    for ii in range(IBUF):
        for j in range(n_docs):
            b = ii * n_docs + j
            pltpu.async_copy(
                table_hbm.at[idx_v.at[ii, j]], rows_v.at[b], sems[b])

    scale = jnp.float32(1.0 / doc_len)

    @pl.loop(0, bpw, step=IBUF)
    def _row(i0):
        for ii in range(IBUF):
            i = i0 + ii
            for j in range(n_docs):
                b = ii * n_docs + j
                pltpu.make_async_copy(
                    table_hbm.at[idx_v.at[i, j]], rows_v.at[b],
                    sems[b]).wait()

                def acc_body(l, accs, _b=b):
                    return tuple(
                        a + rows_v[_b, l, pl.ds(d * LANES, LANES)]
                        for d, a in enumerate(accs))

                accs = lax.fori_loop(
                    0, doc_len, acc_body,
                    tuple(jnp.zeros((LANES,), jnp.float32)
                          for _ in range(NCOL)),
                    unroll=10)
                for d in range(NCOL):
                    out_v[i, j, pl.ds(d * LANES, LANES)] = accs[d] * scale

                nxt = i + IBUF

                @pl.when(nxt < bpw)
                def _prefetch(_b=b, _j=j, _nxt=nxt):
                    pltpu.async_copy(
                        table_hbm.at[idx_v.at[_nxt, _j]], rows_v.at[_b],
                        sems[_b])

    pltpu.sync_copy(out_v, out_hbm.at[pl.ds(base, bpw)])


def kernel(numericalized_doc_toks, embedding):
    batch, n_docs, doc_len = numericalized_doc_toks.shape
    vocab = embedding.shape[0]
    bpw = batch // NW
    nbuf = IBUF * n_docs
    vpad = vocab

    mesh = plsc.VectorSubcoreMesh(core_axis_name="c", subcore_axis_name="s")

    cblk = CBLK
    n_cblk = (vpad + cblk - 1) // cblk
    relayout = pl.pallas_call(
        _tc_transpose_body,
        grid=(n_cblk,),
        in_specs=[pl.BlockSpec((EMBED_DIM, cblk), lambda k: (0, k))],
        out_specs=pl.BlockSpec((cblk // 2, 2 * EMBED_DIM), lambda k: (k, 0)),
        out_shape=jax.ShapeDtypeStruct((vpad // 2, 2 * EMBED_DIM),
                                       jnp.float32),
    )
    table_rm = relayout(embedding.T).reshape(vpad, EMBED_DIM)

    gather = functools.partial(
        pl.kernel,
        out_type=jax.ShapeDtypeStruct((batch, n_docs, EMBED_DIM),
                                      jnp.float32),
        mesh=mesh,
        scratch_types=[
            pltpu.VMEM((bpw, n_docs, doc_len), jnp.int32),
            pltpu.VMEM((nbuf, doc_len, EMBED_DIM), jnp.float32),
            pltpu.VMEM((bpw, n_docs, EMBED_DIM), jnp.float32),
        ] + [pltpu.SemaphoreType.DMA] * nbuf,
        compiler_params=pltpu.CompilerParams(use_tc_tiling_on_sc=False),
    )(_pooled_gather_body)
    return gather(numericalized_doc_toks, table_rm)
